# baseline (speedup 1.0000x reference)
import jax
import jax.numpy as jnp
from jax import lax
from jax.experimental import pallas as pl
from jax.experimental.pallas import tpu as pltpu
from jax.experimental.pallas import tpu_sc as plsc

_NE = 1000000
_BATCH = 16384
_HIST = 50
_FEATURES = 64
_PADF = 128
_N = _BATCH * _HIST
_NC = 2
_NS = 16
_NW = _NC * _NS
_PER_W = _N // _NW
_CHUNK = 128
_NCHUNK = _PER_W // _CHUNK
_NBUF = 3
_BCOLS = _BATCH // _CHUNK


def _embed_body(idx_hbm, idx2_hbm, table_hbm, out_hbm, idx_v, idx2_v,
                rows_v, trans_v, gsem, osem):
    wid = lax.axis_index("s") * _NC + lax.axis_index("c")
    base = wid * _PER_W
    pltpu.sync_copy(idx_hbm.at[pl.ds(base, _PER_W)], idx_v)
    pltpu.sync_copy(idx2_hbm.at[pl.ds(base, _PER_W)], idx2_v)
    lane = lax.iota(jnp.int32, 16)

    def issue_gather(j, s):
        pltpu.async_copy(
            table_hbm.at[idx2_v.at[pl.ds(j * _CHUNK, _CHUNK)]],
            rows_v.at[s], gsem.at[s])

    def wait_gather(s):
        pltpu.make_async_copy(
            table_hbm.at[pl.ds(0, _CHUNK)], rows_v.at[s], gsem.at[s]).wait()

    perms = [(lane + k) & 15 for k in range(16)]

    def transpose(j, s):
        rows2 = rows_v.at[s]
        tr = trans_v.at[s]

        @pl.loop(0, _CHUNK // 16)
        def _(rb):
            rowv = lane + rb * 16
            par = idx_v[pl.ds(j * _CHUNK + rb * 16, 16)]
            parofs = (par & 1) << 6
            for cb in range(_FEATURES // 16):
                basev = parofs + (cb * 16)
                for k in range(16):
                    src_col = perms[k] + basev
                    dst_row = perms[k] + (cb * 16)
                    vals = plsc.load_gather(rows2, [rowv, src_col])
                    plsc.store_scatter(tr, [dst_row, rowv], vals)

    def issue_wb(j, s):
        tc = wid * _NCHUNK + j
        h = tc // _BCOLS
        bcol = tc % _BCOLS
        pltpu.async_copy(
            trans_v.at[s],
            out_hbm.at[h].at[:, pl.ds(bcol * _CHUNK, _CHUNK)],
            osem.at[s])

    def wait_wb(s):
        pltpu.make_async_copy(
            trans_v.at[s], out_hbm.at[0].at[:, pl.ds(0, _CHUNK)],
            osem.at[s]).wait()

    for b in range(_NBUF - 1):
        issue_gather(b, b)

    wait_gather(0)
    transpose(0, 0)
    issue_wb(0, 0)
    issue_gather(_NBUF - 1, _NBUF - 1)

    @pl.loop(1, _NCHUNK - 1, step=_NBUF)
    def _(j0):
        for b in range(_NBUF):
            j = j0 + b
            s = (1 + b) % _NBUF
            sp = b % _NBUF
            wait_gather(s)
            transpose(j, s)
            issue_wb(j, s)

            @pl.when(j < _NCHUNK - _NBUF + 1)
            def _():
                wait_wb(sp)
                issue_gather(j - 1 + _NBUF, sp)

    wait_gather((_NCHUNK - 1) % _NBUF)
    transpose(_NCHUNK - 1, (_NCHUNK - 1) % _NBUF)
    issue_wb(_NCHUNK - 1, (_NCHUNK - 1) % _NBUF)

    for j in range(_NCHUNK - _NBUF, _NCHUNK):
        wait_wb(j % _NBUF)


@jax.jit
def kernel(inputs, embedding):
    idx = inputs.T.reshape(_N)
    idx2 = jax.lax.shift_right_logical(idx, 1)
    table = embedding.reshape(_NE // 2, _PADF)
    out = pl.kernel(
        _embed_body,
        out_type=jax.ShapeDtypeStruct((_HIST, _FEATURES, _BATCH), jnp.float32),
        mesh=plsc.VectorSubcoreMesh(core_axis_name="c", subcore_axis_name="s"),
        compiler_params=pltpu.CompilerParams(
            use_tc_tiling_on_sc=True, needs_layout_passes=False),
        scratch_types=[
            pltpu.VMEM((_PER_W,), jnp.int32),
            pltpu.VMEM((_PER_W,), jnp.int32),
            pltpu.VMEM((_NBUF, _CHUNK, _PADF), jnp.float32),
            pltpu.VMEM((_NBUF, _FEATURES, _CHUNK), jnp.float32),
            pltpu.SemaphoreType.DMA((_NBUF,)),
            pltpu.SemaphoreType.DMA((_NBUF,)),
        ],
    )(idx, idx2, table)
    return out.transpose(2, 0, 1)


# speedup vs baseline: 1.4569x; 1.4569x over previous
import jax
import jax.numpy as jnp
from jax import lax
from jax.experimental import pallas as pl
from jax.experimental.pallas import tpu as pltpu
from jax.experimental.pallas import tpu_sc as plsc

_NE = 1000000
_BATCH = 16384
_HIST = 50
_FEATURES = 64
_PADF = 128
_N = _BATCH * _HIST
_NC = 2
_NS = 16
_NW = _NC * _NS
_PER_W = _N // _NW
_CHUNK = 128
_NCHUNK = _PER_W // _CHUNK
_NBUF = 3
_BCOLS = _BATCH // _CHUNK


def _embed_body(idx_hbm, table_hbm, out_hbm, idx_v, rows_v, trans_v,
                gsem, osem):
    wid = lax.axis_index("s") * _NC + lax.axis_index("c")
    base = wid * _PER_W
    pltpu.sync_copy(idx_hbm.at[pl.ds(base, _PER_W)], idx_v)
    lane = lax.iota(jnp.int32, 16)

    def issue_gather(j, s):
        pltpu.async_copy(
            table_hbm.at[idx_v.at[pl.ds(j * _CHUNK, _CHUNK)]],
            rows_v.at[s], gsem.at[s])

    def wait_gather(s):
        pltpu.make_async_copy(
            table_hbm.at[pl.ds(0, _CHUNK)], rows_v.at[s], gsem.at[s]).wait()

    perms = [(lane + k) & 15 for k in range(16)]

    def transpose(s):
        rows2 = rows_v.at[s]
        tr = trans_v.at[s]

        @pl.loop(0, _CHUNK // 16)
        def _(rb):
            rowv = lane + rb * 16
            for cb in range(_FEATURES // 16):
                c0 = cb * 16
                cols = [perms[k] + c0 for k in range(16)]
                vals = [plsc.load_gather(rows2, [rowv, cols[k]])
                        for k in range(16)]
                for k in range(16):
                    plsc.store_scatter(tr, [cols[k], rowv], vals[k])

    def issue_wb(j, s):
        tc = wid * _NCHUNK + j
        h = tc // _BCOLS
        bcol = tc % _BCOLS
        pltpu.async_copy(
            trans_v.at[s],
            out_hbm.at[h].at[:, pl.ds(bcol * _CHUNK, _CHUNK)],
            osem.at[s])

    def wait_wb(s):
        pltpu.make_async_copy(
            trans_v.at[s], out_hbm.at[0].at[:, pl.ds(0, _CHUNK)],
            osem.at[s]).wait()

    for b in range(_NBUF - 1):
        issue_gather(b, b)

    wait_gather(0)
    transpose(0)
    issue_wb(0, 0)
    issue_gather(_NBUF - 1, _NBUF - 1)

    @pl.loop(1, _NCHUNK - 1, step=_NBUF)
    def _(j0):
        for b in range(_NBUF):
            j = j0 + b
            s = (1 + b) % _NBUF
            sp = b % _NBUF
            wait_gather(s)
            transpose(s)
            issue_wb(j, s)

            @pl.when(j < _NCHUNK - _NBUF + 1)
            def _():
                wait_wb(sp)
                issue_gather(j - 1 + _NBUF, sp)

    wait_gather((_NCHUNK - 1) % _NBUF)
    transpose((_NCHUNK - 1) % _NBUF)
    issue_wb(_NCHUNK - 1, (_NCHUNK - 1) % _NBUF)

    for j in range(_NCHUNK - _NBUF, _NCHUNK):
        wait_wb(j % _NBUF)


@jax.jit
def kernel(inputs, embedding):
    idx = inputs.T.reshape(_N)
    table = jnp.concatenate(
        [embedding, jnp.zeros((_NE, _PADF - _FEATURES), jnp.float32)], axis=1)
    out = pl.kernel(
        _embed_body,
        out_type=jax.ShapeDtypeStruct((_HIST, _FEATURES, _BATCH), jnp.float32),
        mesh=plsc.VectorSubcoreMesh(core_axis_name="c", subcore_axis_name="s"),
        compiler_params=pltpu.CompilerParams(
            use_tc_tiling_on_sc=True, needs_layout_passes=False),
        scratch_types=[
            pltpu.VMEM((_PER_W,), jnp.int32),
            pltpu.VMEM((_NBUF, _CHUNK, _PADF), jnp.float32),
            pltpu.VMEM((_NBUF, _FEATURES, _CHUNK), jnp.float32),
            pltpu.SemaphoreType.DMA((_NBUF,)),
            pltpu.SemaphoreType.DMA((_NBUF,)),
        ],
    )(idx, table)
    return out.transpose(2, 0, 1)
